# baseline (speedup 1.0000x reference)
import functools

import jax
import jax.numpy as jnp
from jax import lax
from jax.experimental import pallas as pl
from jax.experimental.pallas import tpu as pltpu
from jax.experimental.pallas import tpu_sc as plsc

_SEED_ROWS = 64


def _sc_seed(logs, V):
    info = plsc.get_sparse_core_info()
    NC, NS = info.num_cores, info.num_subcores
    NW = NC * NS
    rows_per_w = _SEED_ROWS // NW

    mesh = plsc.VectorSubcoreMesh(core_axis_name="c", subcore_axis_name="s")

    @functools.partial(
        pl.kernel,
        mesh=mesh,
        out_type=jax.ShapeDtypeStruct((_SEED_ROWS, V), jnp.float32),
        scratch_types=[
            pltpu.VMEM((V,), jnp.float32),
            pltpu.SemaphoreType.DMA,
        ],
    )
    def seed_kernel(logs_hbm, seed_hbm, row_v, sem):
        wid = lax.axis_index("s") * NC + lax.axis_index("c")
        pltpu.sync_copy(logs_hbm.at[pl.ds(0, V)], row_v)
        base = wid * rows_per_w
        copies = [
            pltpu.make_async_copy(row_v, seed_hbm.at[base + i], sem)
            for i in range(rows_per_w)
        ]
        for c in copies:
            c.start()
        for c in copies:
            c.wait()

    return seed_kernel(logs)


def _tc_expand(seed, B, V):
    nblk = B // _SEED_ROWS

    def body(seed_vmem, out_hbm, sem):
        copies = [
            pltpu.make_async_copy(
                seed_vmem, out_hbm.at[pl.ds(k * _SEED_ROWS, _SEED_ROWS)], sem
            )
            for k in range(nblk)
        ]
        for c in copies:
            c.start()
        for c in copies:
            c.wait()

    return pl.pallas_call(
        body,
        in_specs=[pl.BlockSpec(memory_space=pltpu.VMEM)],
        out_specs=pl.BlockSpec(memory_space=pltpu.MemorySpace.HBM),
        out_shape=jax.ShapeDtypeStruct((B, V), jnp.float32),
        scratch_shapes=[pltpu.SemaphoreType.DMA],
    )(seed)


def kernel(hist, idx, logs):
    B = hist.shape[1]
    V = logs.shape[0] - 1
    seed = _sc_seed(logs, V)
    return _tc_expand(seed, B, V)


# speedup vs baseline: 2.7168x; 2.7168x over previous
import jax
import jax.numpy as jnp
from jax.experimental import pallas as pl
from jax.experimental.pallas import tpu as pltpu


def kernel(hist, idx, logs):
    B = hist.shape[1]
    V = logs.shape[0] - 1
    VT = 1024

    def body(lg_ref, out_ref):
        out_ref[...] = jnp.broadcast_to(lg_ref[...], out_ref.shape)

    logs_col = logs[:V].reshape(V, 1)
    out_t = pl.pallas_call(
        body,
        grid=(pl.cdiv(V, VT),),
        in_specs=[pl.BlockSpec((VT, 1), lambda i: (i, 0))],
        out_specs=pl.BlockSpec((VT, B), lambda i: (i, 0)),
        out_shape=jax.ShapeDtypeStruct((V, B), jnp.float32),
    )(logs_col)
    return out_t.T


# speedup vs baseline: 3.0598x; 1.1262x over previous
import functools

import jax
import jax.numpy as jnp
from jax import lax
from jax.experimental import pallas as pl
from jax.experimental.pallas import tpu as pltpu
from jax.experimental.pallas import tpu_sc as plsc

_RB = 16


def kernel(hist, idx, logs):
    B = hist.shape[1]
    V = logs.shape[0] - 1

    info = plsc.get_sparse_core_info()
    NC, NS, L = info.num_cores, info.num_subcores, info.num_lanes
    NW = NC * NS
    CH = -(-V // NW)
    CH = -(-CH // (2 * _RB)) * (2 * _RB)
    npair = CH // (2 * _RB)

    mesh = plsc.VectorSubcoreMesh(core_axis_name="c", subcore_axis_name="s")

    @functools.partial(
        pl.kernel,
        mesh=mesh,
        out_type=jax.ShapeDtypeStruct((V, B), jnp.float32),
        scratch_types=[
            pltpu.VMEM((CH,), jnp.float32),
            pltpu.VMEM((_RB, B), jnp.float32),
            pltpu.VMEM((_RB, B), jnp.float32),
            pltpu.SemaphoreType.DMA,
            pltpu.SemaphoreType.DMA,
        ],
        compiler_params=pltpu.CompilerParams(needs_layout_passes=False),
    )
    def bcast_t(logs_hbm, out_hbm, lg_v, buf0, buf1, sem0, sem1):
        wid = lax.axis_index("s") * NC + lax.axis_index("c")
        base = jnp.minimum(wid * CH, V - CH)
        pltpu.sync_copy(logs_hbm.at[pl.ds(base, CH)], lg_v)
        bufs = (buf0, buf1)
        sems = (sem0, sem1)

        col_idx = [lax.iota(jnp.int32, L) + j * L for j in range(B // L)]

        def fill(buf, ch):
            def row_body(r, carry):
                idx = jnp.full((L,), ch * _RB + r, jnp.int32)
                vec = plsc.load_gather(lg_v, [idx])
                row_idx = jnp.full((L,), r, jnp.int32)
                for j in range(B // L):
                    plsc.store_scatter(buf, [row_idx, col_idx[j]], vec)
                return carry

            lax.fori_loop(0, _RB, row_body, 0)

        def start(p, ch):
            cp = pltpu.make_async_copy(
                bufs[p], out_hbm.at[pl.ds(base + ch * _RB, _RB)], sems[p]
            )
            cp.start()

        def drain(p, ch):
            pltpu.make_async_copy(
                bufs[p], out_hbm.at[pl.ds(base + ch * _RB, _RB)], sems[p]
            ).wait()

        for p in range(2):
            fill(bufs[p], p)
            start(p, p)

        def step(k2, carry):
            for p in range(2):
                ch = k2 * 2 + p
                drain(p, ch)
                fill(bufs[p], ch)
                start(p, ch)
            return carry

        lax.fori_loop(1, npair, step, 0)
        for p in range(2):
            drain(p, p)

    out_t = bcast_t(logs)
    return out_t.T
